# baseline (speedup 1.0000x reference)
import jax
import jax.numpy as jnp
from jax import lax
from jax.experimental import pallas as pl

N_ROI = 5050
N_PAD = 5120
N_ROWS = N_PAD // 128
N_GT = 50
N_SAMPLE = 128
POS_NUM = 32
NEG_NUM = 96
LOC_MEAN = (0.0, 0.0, 0.0, 0.0)
LOC_STD = (0.1, 0.1, 0.2, 0.2)


def _iou_body(coords_ref, gt_ref, mx_ref, mi_ref):
    rx1 = coords_ref[0 * N_ROWS:1 * N_ROWS, :]
    ry1 = coords_ref[1 * N_ROWS:2 * N_ROWS, :]
    rx2 = coords_ref[2 * N_ROWS:3 * N_ROWS, :]
    ry2 = coords_ref[3 * N_ROWS:4 * N_ROWS, :]
    area_r = (rx2 - rx1) * (ry2 - ry1)
    best = jnp.full((N_ROWS, 128), -1.0, jnp.float32)
    bidx = jnp.zeros((N_ROWS, 128), jnp.int32)
    for g in range(N_GT):
        gx1 = gt_ref[0, g]
        gy1 = gt_ref[1, g]
        gx2 = gt_ref[2, g]
        gy2 = gt_ref[3, g]
        ab = (gx2 - gx1) * (gy2 - gy1)
        wx = jnp.maximum(jnp.minimum(rx2, gx2) - jnp.maximum(rx1, gx1), 0.0)
        wy = jnp.maximum(jnp.minimum(ry2, gy2) - jnp.maximum(ry1, gy1), 0.0)
        inter = wx * wy
        iou = inter / ((area_r + ab) - inter)
        p = iou > best
        best = jnp.where(p, iou, best)
        bidx = jnp.where(p, g, bidx)
    mx_ref[...] = best
    mi_ref[...] = bidx


def _sel_body(kp_ref, kn_ref, coords_ref, iidx_ref, gt_ref, src_ref, off_ref, lab_ref):
    lin = (lax.broadcasted_iota(jnp.int32, (N_ROWS, 128), 0) * 128
           + lax.broadcasted_iota(jnp.int32, (N_ROWS, 128), 1))
    col1 = lax.broadcasted_iota(jnp.int32, (1, 128), 1)

    def argmin_step(m, carry):
        keys, keepv = carry
        mn = jnp.min(keys)
        sel = jnp.min(jnp.where(keys == mn, lin, N_PAD))
        keepv = jnp.where(col1 == m, sel, keepv)
        keys = jnp.where(lin == sel, jnp.inf, keys)
        return keys, keepv

    keepv = jnp.zeros((1, 128), jnp.int32)
    _, keepv = lax.fori_loop(0, POS_NUM, argmin_step, (kp_ref[...], keepv))
    _, keepv = lax.fori_loop(POS_NUM, N_SAMPLE, argmin_step, (kn_ref[...], keepv))

    gtrows = [gt_ref[pl.ds(r, 1), :] for r in range(5)]
    sacc = [jnp.zeros((1, 128), jnp.float32) for _ in range(4)]
    dacc = [jnp.zeros((1, 128), jnp.float32) for _ in range(4)]
    lacc = jnp.zeros((1, 128), jnp.float32)
    for m in range(N_SAMPLE):
        sel = jnp.sum(jnp.where(col1 == m, keepv, 0))
        row = sel // 128
        col = sel % 128
        cmask = col1 == col
        giv = iidx_ref[pl.ds(row, 1), :]
        gidx = jnp.sum(jnp.where(cmask, giv, 0))
        gmask = col1 == gidx
        mput = col1 == m
        for r in range(4):
            rv = coords_ref[pl.ds(r * N_ROWS + row, 1), :]
            sacc[r] = jnp.where(mput, jnp.sum(jnp.where(cmask, rv, 0.0)), sacc[r])
            dacc[r] = jnp.where(mput, jnp.sum(jnp.where(gmask, gtrows[r], 0.0)), dacc[r])
        lacc = jnp.where(mput, jnp.sum(jnp.where(gmask, gtrows[4], 0.0)), lacc)

    sx1, sy1, sx2, sy2 = sacc
    dx1, dy1, dx2, dy2 = dacc
    pw = sx2 - sx1
    ph = sy2 - sy1
    px = sx1 + 0.5 * pw
    py = sy1 + 0.5 * ph
    gw = dx2 - dx1
    gh = dy2 - dy1
    gx = dx1 + 0.5 * gw
    gy = dy1 + 0.5 * gh
    offs = [(gx - px) / pw, (gy - py) / ph, jnp.log(gw / pw), jnp.log(gh / ph)]
    for r in range(4):
        src_ref[pl.ds(r, 1), :] = sacc[r]
        off_ref[pl.ds(r, 1), :] = (offs[r] - LOC_MEAN[r]) / LOC_STD[r]
    lab_ref[...] = jnp.where(col1 < POS_NUM, lacc.astype(jnp.int32) + 1, 0)


def kernel(roi, gt_bbox, gt_label):
    roi_all = jnp.concatenate([roi, gt_bbox], axis=0)
    pad_rows = jnp.tile(jnp.array([[0.0, 0.0, 1.0, 1.0]], jnp.float32), (N_PAD - N_ROI, 1))
    coords = jnp.concatenate([roi_all, pad_rows], axis=0).T.reshape(4 * N_ROWS, 128)
    gt_pack = jnp.zeros((8, 128), jnp.float32)
    gt_pack = gt_pack.at[0:4, 0:N_GT].set(gt_bbox.T)
    gt_pack = gt_pack.at[4, 0:N_GT].set(gt_label.astype(jnp.float32))

    iou_max, iou_idx = pl.pallas_call(
        _iou_body,
        out_shape=[
            jax.ShapeDtypeStruct((N_ROWS, 128), jnp.float32),
            jax.ShapeDtypeStruct((N_ROWS, 128), jnp.int32),
        ],
    )(coords, gt_pack)
    iou_max_f = iou_max.reshape(-1)[:N_ROI]

    kp, kn = jax.random.split(jax.random.key(1))

    def keys_for(mask, key, max_num):
        cnt = jnp.sum(mask).astype(jnp.int32)
        mi = mask.astype(jnp.int32)
        rank = jnp.cumsum(mi) - mi
        i_f = jnp.arange(N_ROI, dtype=jnp.float32)
        w_full = jnp.where(mask, i_f, 0.0)
        w_c = jnp.zeros((N_ROI + 1,), jnp.float32).at[
            jnp.where(mask, rank, N_ROI)].set(w_full)[:N_ROI]
        s = jnp.sum(w_c)
        gum = jax.random.gumbel(key, (N_ROI,), jnp.float32)
        g_full = -gum[rank] - jnp.log(w_full / s)
        g_full = jnp.where(mask, g_full, jnp.inf)
        key_alt = jnp.where(mask, rank.astype(jnp.float32), jnp.inf)
        return jnp.where(cnt > max_num, g_full, key_alt)

    key_pos = keys_for(iou_max_f >= 0.5, kp, POS_NUM)
    key_neg = keys_for(iou_max_f <= 0.5, kn, NEG_NUM)
    padk = jnp.full((N_PAD - N_ROI,), jnp.inf, jnp.float32)
    key_pos = jnp.concatenate([key_pos, padk]).reshape(N_ROWS, 128)
    key_neg = jnp.concatenate([key_neg, padk]).reshape(N_ROWS, 128)

    src, off, lab = pl.pallas_call(
        _sel_body,
        out_shape=[
            jax.ShapeDtypeStruct((4, 128), jnp.float32),
            jax.ShapeDtypeStruct((4, 128), jnp.float32),
            jax.ShapeDtypeStruct((1, 128), jnp.int32),
        ],
    )(key_pos, key_neg, coords, iou_idx, gt_pack)

    return src.T, off.T, lab.reshape(-1)


# speedup vs baseline: 1.3048x; 1.3048x over previous
import functools
import jax
import jax.numpy as jnp
from jax import lax
from jax.experimental import pallas as pl
from jax.experimental.pallas import tpu as pltpu, tpu_sc as plsc

N_ROI = 5050
N_PAD = 5120
N_ROWS = N_PAD // 128
N_GT = 50
N_SAMPLE = 128
POS_NUM = 32
NEG_NUM = 96
NW = 32
PER_W = N_PAD // NW
CHUNKS = PER_W // 16
LN2 = 0.6931471805599453


def _sc_iou_body(cx1, cy1, cx2, cy2, gt_hbm, mx_hbm, mi_hbm,
                 v1, v2, v3, v4, gv, mxv, miv):
    wid = lax.axis_index("s") * 2 + lax.axis_index("c")
    base = wid * PER_W
    for ch, v in zip((cx1, cy1, cx2, cy2), (v1, v2, v3, v4)):
        pltpu.sync_copy(ch.at[pl.ds(base, PER_W)], v)
    pltpu.sync_copy(gt_hbm, gv)

    gchunk = [[gv[pl.ds(r * 64 + k * 16, 16)] for k in range(4)]
              for r in range(4)]

    def chunk(c, carry):
        rx1 = v1[pl.ds(c * 16, 16)]
        ry1 = v2[pl.ds(c * 16, 16)]
        rx2 = v3[pl.ds(c * 16, 16)]
        ry2 = v4[pl.ds(c * 16, 16)]
        area_r = (rx2 - rx1) * (ry2 - ry1)
        best = jnp.full((16,), -1.0, jnp.float32)
        bidx = jnp.zeros((16,), jnp.int32)
        for g in range(N_GT):
            gx1 = gchunk[0][g // 16][g % 16]
            gy1 = gchunk[1][g // 16][g % 16]
            gx2 = gchunk[2][g // 16][g % 16]
            gy2 = gchunk[3][g // 16][g % 16]
            ab = (gx2 - gx1) * (gy2 - gy1)
            wx = jnp.maximum(jnp.minimum(rx2, gx2) - jnp.maximum(rx1, gx1), 0.0)
            wy = jnp.maximum(jnp.minimum(ry2, gy2) - jnp.maximum(ry1, gy1), 0.0)
            inter = wx * wy
            iou = inter / ((area_r + ab) - inter)
            p = iou > best
            best = jnp.where(p, iou, best)
            bidx = jnp.where(p, g, bidx)
        mxv[pl.ds(c * 16, 16)] = best
        miv[pl.ds(c * 16, 16)] = bidx
        return carry

    lax.fori_loop(0, CHUNKS, chunk, 0)
    pltpu.sync_copy(mxv, mx_hbm.at[pl.ds(base, PER_W)])
    pltpu.sync_copy(miv, mi_hbm.at[pl.ds(base, PER_W)])


def _sel_body(kp_ref, kn_ref, keep_ref):
    rowi = lax.broadcasted_iota(jnp.int32, (N_ROWS, 128), 0)
    lin = rowi * 128 + lax.broadcasted_iota(jnp.int32, (N_ROWS, 128), 1)
    col1 = lax.broadcasted_iota(jnp.int32, (1, 128), 1)

    def argmin_step(m, carry):
        keys, keepv = carry
        rowmin = jnp.min(keys, axis=0, keepdims=True)
        rowarg = jnp.min(jnp.where(keys == rowmin, rowi, N_ROWS), axis=0,
                         keepdims=True)
        mn = jnp.min(rowmin)
        sel = jnp.min(jnp.where(rowmin == mn, rowarg * 128 + col1, N_PAD))
        keepv = jnp.where(col1 == m, sel, keepv)
        keys = jnp.where(lin == sel, jnp.inf, keys)
        return keys, keepv

    keepv = jnp.zeros((1, 128), jnp.int32)
    _, keepv = lax.fori_loop(0, POS_NUM, argmin_step, (kp_ref[...], keepv))
    _, keepv = lax.fori_loop(POS_NUM, N_SAMPLE, argmin_step, (kn_ref[...], keepv))
    keep_ref[...] = keepv


def _log_poly(t):
    bits = lax.bitcast_convert_type(t, jnp.int32)
    e = ((bits >> 23) & 0xFF) - 126
    m = lax.bitcast_convert_type((bits & 0x7FFFFF) | 0x3F000000, jnp.float32)
    s = (m - 1.0) / (m + 1.0)
    s2 = s * s
    p = s * (2.0 + s2 * (2.0 / 3.0 + s2 * (2.0 / 5.0
             + s2 * (2.0 / 7.0 + s2 * (2.0 / 9.0)))))
    return e.astype(jnp.float32) * LN2 + p


def _sc_gather_body(cx1, cy1, cx2, cy2, iidx_h, gx1_h, gy1_h, gx2_h, gy2_h,
                    glab_h, keep_h, src_h, off_h, lab_h,
                    kv, iv, s1, s2, s3, s4, gi, d1, d2, d3, d4, gl,
                    so, oo, lo, sem):
    wid = lax.axis_index("s") * 2 + lax.axis_index("c")

    @pl.when(wid < 8)
    def _():
        pltpu.sync_copy(keep_h, kv)
        base = wid * 16
        iv[...] = kv[pl.ds(base, 16)]
        copies = [pltpu.async_copy(h.at[iv], d, sem)
                  for h, d in ((cx1, s1), (cy1, s2), (cx2, s3), (cy2, s4),
                               (iidx_h, gi))]
        for c in copies:
            c.wait()
        giv = gi[...]
        copies = [pltpu.async_copy(h.at[giv], d, sem)
                  for h, d in ((gx1_h, d1), (gy1_h, d2), (gx2_h, d3),
                               (gy2_h, d4), (glab_h, gl))]
        for c in copies:
            c.wait()

        sx1, sy1, sx2, sy2 = s1[...], s2[...], s3[...], s4[...]
        dx1, dy1, dx2, dy2 = d1[...], d2[...], d3[...], d4[...]
        pw = sx2 - sx1
        ph = sy2 - sy1
        px = sx1 + 0.5 * pw
        py = sy1 + 0.5 * ph
        gw = dx2 - dx1
        gh = dy2 - dy1
        gx = dx1 + 0.5 * gw
        gy = dy1 + 0.5 * gh
        so[pl.ds(0, 16)] = sx1
        so[pl.ds(16, 16)] = sy1
        so[pl.ds(32, 16)] = sx2
        so[pl.ds(48, 16)] = sy2
        oo[pl.ds(0, 16)] = ((gx - px) / pw) / 0.1
        oo[pl.ds(16, 16)] = ((gy - py) / ph) / 0.1
        oo[pl.ds(32, 16)] = _log_poly(gw / pw) / 0.2
        oo[pl.ds(48, 16)] = _log_poly(gh / ph) / 0.2
        widv = jnp.zeros((16,), jnp.int32) + wid
        posm = jnp.minimum(jnp.maximum(2 - widv, 0), 1)
        lo[...] = (gl[...] + 1) * posm
        for r in range(4):
            pltpu.sync_copy(so.at[pl.ds(r * 16, 16)],
                            src_h.at[pl.ds(r * 128 + base, 16)])
            pltpu.sync_copy(oo.at[pl.ds(r * 16, 16)],
                            off_h.at[pl.ds(r * 128 + base, 16)])
        pltpu.sync_copy(lo, lab_h.at[pl.ds(base, 16)])


def kernel(roi, gt_bbox, gt_label):
    mesh = plsc.VectorSubcoreMesh(core_axis_name="c", subcore_axis_name="s")

    roi_all = jnp.concatenate([roi, gt_bbox], axis=0)
    pad_rows = jnp.tile(jnp.array([[0.0, 0.0, 1.0, 1.0]], jnp.float32),
                        (N_PAD - N_ROI, 1))
    coords4 = jnp.concatenate([roi_all, pad_rows], axis=0).T
    cx1, cy1, cx2, cy2 = (coords4[r] for r in range(4))
    gt_flat = jnp.zeros((4, 64), jnp.float32).at[:, :N_GT].set(
        gt_bbox.T).reshape(-1)
    gtc = [jnp.zeros((64,), jnp.float32).at[:N_GT].set(gt_bbox[:, r])
           for r in range(4)]
    glab = jnp.zeros((64,), jnp.int32).at[:N_GT].set(gt_label.astype(jnp.int32))

    sc_iou = functools.partial(
        pl.kernel,
        mesh=mesh,
        out_type=[
            jax.ShapeDtypeStruct((N_PAD,), jnp.float32),
            jax.ShapeDtypeStruct((N_PAD,), jnp.int32),
        ],
        scratch_types=[
            pltpu.VMEM((PER_W,), jnp.float32),
            pltpu.VMEM((PER_W,), jnp.float32),
            pltpu.VMEM((PER_W,), jnp.float32),
            pltpu.VMEM((PER_W,), jnp.float32),
            pltpu.VMEM((256,), jnp.float32),
            pltpu.VMEM((PER_W,), jnp.float32),
            pltpu.VMEM((PER_W,), jnp.int32),
        ],
    )(_sc_iou_body)
    iou_max, iou_idx = sc_iou(cx1, cy1, cx2, cy2, gt_flat)
    iou_max_f = iou_max[:N_ROI]

    kp, kn = jax.random.split(jax.random.key(1))

    def keys_for(mask, key, max_num):
        cnt = jnp.sum(mask).astype(jnp.int32)
        mi = mask.astype(jnp.int32)
        rank = jnp.cumsum(mi) - mi
        i_f = jnp.arange(N_ROI, dtype=jnp.float32)
        w_full = jnp.where(mask, i_f, 0.0)
        s = jnp.sum(w_full)
        gum = jax.random.gumbel(key, (N_ROI,), jnp.float32)
        g_full = -gum[rank] - jnp.log(w_full / s)
        g_full = jnp.where(mask, g_full, jnp.inf)
        key_alt = jnp.where(mask, rank.astype(jnp.float32), jnp.inf)
        return jnp.where(cnt > max_num, g_full, key_alt)

    key_pos = keys_for(iou_max_f >= 0.5, kp, POS_NUM)
    key_neg = keys_for(iou_max_f <= 0.5, kn, NEG_NUM)
    padk = jnp.full((N_PAD - N_ROI,), jnp.inf, jnp.float32)
    key_pos = jnp.concatenate([key_pos, padk]).reshape(N_ROWS, 128)
    key_neg = jnp.concatenate([key_neg, padk]).reshape(N_ROWS, 128)

    keep = pl.pallas_call(
        _sel_body,
        out_shape=jax.ShapeDtypeStruct((1, 128), jnp.int32),
    )(key_pos, key_neg).reshape(-1)

    sc_gather = functools.partial(
        pl.kernel,
        mesh=mesh,
        out_type=[
            jax.ShapeDtypeStruct((512,), jnp.float32),
            jax.ShapeDtypeStruct((512,), jnp.float32),
            jax.ShapeDtypeStruct((128,), jnp.int32),
        ],
        scratch_types=[
            pltpu.VMEM((128,), jnp.int32),
            pltpu.VMEM((16,), jnp.int32),
            pltpu.VMEM((16,), jnp.float32),
            pltpu.VMEM((16,), jnp.float32),
            pltpu.VMEM((16,), jnp.float32),
            pltpu.VMEM((16,), jnp.float32),
            pltpu.VMEM((16,), jnp.int32),
            pltpu.VMEM((16,), jnp.float32),
            pltpu.VMEM((16,), jnp.float32),
            pltpu.VMEM((16,), jnp.float32),
            pltpu.VMEM((16,), jnp.float32),
            pltpu.VMEM((16,), jnp.int32),
            pltpu.VMEM((64,), jnp.float32),
            pltpu.VMEM((64,), jnp.float32),
            pltpu.VMEM((16,), jnp.int32),
            pltpu.SemaphoreType.DMA,
        ],
    )(_sc_gather_body)
    src, off, lab = sc_gather(cx1, cy1, cx2, cy2, iou_idx,
                              gtc[0], gtc[1], gtc[2], gtc[3], glab, keep)

    return src.reshape(4, 128).T, off.reshape(4, 128).T, lab
